# baseline (speedup 1.0000x reference)
import functools

import jax
import jax.numpy as jnp
from jax import lax
from jax.experimental import pallas as pl
from jax.experimental.pallas import tpu as pltpu
from jax.experimental.pallas import tpu_sc as plsc

NC = 2
NS = 16
NW = NC * NS

LANE = 128
RING = 8


def _windowed_rowsum_tc(table):
    v, d = table.shape
    t128 = table.reshape(v * d // LANE, LANE)
    rows = t128.shape[0]
    blk = 2000
    assert rows % blk == 0

    def body(t_ref, o_ref):
        ii = lax.broadcasted_iota(jnp.int32, (LANE, LANE), 0)
        jj = lax.broadcasted_iota(jnp.int32, (LANE, LANE), 1)
        w = ((ii // d) == (jj // d)).astype(jnp.float32)
        o_ref[...] = jax.lax.dot_general(
            t_ref[...], w, (((1,), (0,)), ((), ())),
            preferred_element_type=jnp.float32)

    out = pl.pallas_call(
        body,
        grid=(rows // blk,),
        in_specs=[pl.BlockSpec((blk, LANE), lambda i: (i, 0))],
        out_specs=pl.BlockSpec((blk, LANE), lambda i: (i, 0)),
        out_shape=jax.ShapeDtypeStruct((rows, LANE), jnp.float32),
    )(t128)
    return out.reshape(v * d)


def _split_shift_tc(indices, shift, s_a, s_b):
    b, s = indices.shape
    rblk = 128
    assert b % rblk == 0

    def body(i_ref, a_ref, b_ref):
        x = i_ref[...] << shift
        a_ref[...] = x[:, :s_a]
        b_ref[...] = jnp.concatenate(
            [x[:, s_a:s], jnp.zeros((rblk, LANE - s_b), jnp.int32)], axis=1)

    return pl.pallas_call(
        body,
        grid=(b // rblk,),
        in_specs=[pl.BlockSpec((rblk, s), lambda i: (i, 0))],
        out_specs=[pl.BlockSpec((rblk, LANE), lambda i: (i, 0)),
                   pl.BlockSpec((rblk, LANE), lambda i: (i, 0))],
        out_shape=[jax.ShapeDtypeStruct((b, LANE), jnp.int32),
                   jax.ShapeDtypeStruct((b, LANE), jnp.int32)],
    )(indices)


def _gather_sc(idx_a, idx_b, rsflat, s_b):
    n_rows = idx_a.shape[0]
    assert n_rows % NW == 0
    n_ch = n_rows // NW

    mesh = plsc.VectorSubcoreMesh(core_axis_name="c", subcore_axis_name="s")

    @functools.partial(
        pl.kernel,
        mesh=mesh,
        out_type=[jax.ShapeDtypeStruct((n_rows, LANE), jnp.float32),
                  jax.ShapeDtypeStruct((n_rows, LANE), jnp.float32)],
        scratch_types=[
            pltpu.VMEM((n_ch, LANE), jnp.int32),
            pltpu.VMEM((n_ch, LANE), jnp.int32),
            pltpu.VMEM((n_ch, LANE), jnp.float32),
            pltpu.VMEM((n_ch, LANE), jnp.float32),
            pltpu.SemaphoreType.DMA,
        ],
    )
    def k(ia_hbm, ib_hbm, rs_hbm, oa_hbm, ob_hbm, ia_v, ib_v, va_v, vb_v, sem):
        wid = lax.axis_index("s") * NC + lax.axis_index("c")
        row0 = wid * n_ch
        pltpu.sync_copy(ia_hbm.at[pl.ds(row0, n_ch)], ia_v)
        pltpu.sync_copy(ib_hbm.at[pl.ds(row0, n_ch)], ib_v)

        def start_a(c):
            pltpu.make_async_copy(
                rs_hbm.at[ia_v.at[c]], va_v.at[c], sem).start()

        def wait_a(c):
            pltpu.make_async_copy(
                rs_hbm.at[ia_v.at[c]], va_v.at[c], sem).wait()

        def start_b(c):
            pltpu.make_async_copy(
                rs_hbm.at[ib_v.at[c, pl.ds(0, s_b)]],
                vb_v.at[c, pl.ds(0, s_b)], sem).start()

        def wait_b(c):
            pltpu.make_async_copy(
                rs_hbm.at[ib_v.at[c, pl.ds(0, s_b)]],
                vb_v.at[c, pl.ds(0, s_b)], sem).wait()

        def ring(start, wait):
            def prime(c, carry):
                start(c)
                return carry
            lax.fori_loop(0, RING, prime, 0)

            def step(c, carry):
                start(c + RING)
                wait(c)
                return carry
            lax.fori_loop(0, n_ch - RING, step, 0)

            def drain(c, carry):
                wait(c)
                return carry
            lax.fori_loop(n_ch - RING, n_ch, drain, 0)

        ring(start_a, wait_a)
        ring(start_b, wait_b)

        pltpu.sync_copy(va_v, oa_hbm.at[pl.ds(row0, n_ch)])
        pltpu.sync_copy(vb_v, ob_hbm.at[pl.ds(row0, n_ch)])

    return k(idx_a, idx_b, rsflat)


def _stitch_mul_tc(g_a, g_b, weights, s_a, s_b):
    b, s = weights.shape
    rblk = 128

    def body(a_ref, b_ref, w_ref, o_ref):
        g = jnp.concatenate([a_ref[...], b_ref[:, :s_b]], axis=1)
        o_ref[...] = g * w_ref[...]

    return pl.pallas_call(
        body,
        grid=(b // rblk,),
        in_specs=[
            pl.BlockSpec((rblk, LANE), lambda i: (i, 0)),
            pl.BlockSpec((rblk, LANE), lambda i: (i, 0)),
            pl.BlockSpec((rblk, s), lambda i: (i, 0)),
        ],
        out_specs=pl.BlockSpec((rblk, s), lambda i: (i, 0)),
        out_shape=jax.ShapeDtypeStruct((b, s), jnp.float32),
    )(g_a, g_b, weights)


def kernel(indices, weights, table):
    b, s, n = indices.shape
    assert n == 1 and b % NW == 0
    s_a = LANE
    s_b = s - s_a
    rsflat = _windowed_rowsum_tc(table)
    idx_a, idx_b = _split_shift_tc(
        indices.reshape(b, s).astype(jnp.int32), 5, s_a, s_b)
    g_a, g_b = _gather_sc(idx_a, idx_b, rsflat, s_b)
    return _stitch_mul_tc(g_a, g_b, weights.reshape(b, s), s_a, s_b)
